# baseline (speedup 1.0000x reference)
import functools

import jax
import jax.numpy as jnp
from jax import lax
from jax.experimental import pallas as pl
from jax.experimental.pallas import tpu as pltpu
from jax.experimental.pallas import tpu_sc as plsc

TEAM = 5
D = 32
NIDS = 2 * TEAM
FD = 96


def _sc_gather(ids3, tabs, ab_tab):
    n = ids3.shape[0] // 3
    info = plsc.get_sparse_core_info()
    nw = info.num_cores * info.num_subcores
    b_per_w = n // nw
    ch = 128
    n_ch = b_per_w // ch

    mesh = plsc.VectorSubcoreMesh(core_axis_name="c", subcore_axis_name="s")

    def body(idx_hbm, tab_hbm, ab_hbm, out_hbm, ab_out,
             idx_v, c_v, s_v, w_v, a_v, sem):
        wid = lax.axis_index("s") * info.num_cores + lax.axis_index("c")
        base = wid * b_per_w
        for t in range(3):
            pltpu.sync_copy(idx_hbm.at[pl.ds(t * n + base, b_per_w)],
                            idx_v.at[pl.ds(t * b_per_w, b_per_w)])

        def chunk(c, carry):
            off = c * ch
            cp1 = pltpu.async_copy(
                tab_hbm.at[idx_v.at[pl.ds(off, ch)]],
                c_v.at[pl.ds(off, ch)], sem)
            cp2 = pltpu.async_copy(
                tab_hbm.at[idx_v.at[pl.ds(b_per_w + off, ch)]],
                s_v.at[pl.ds(off, ch)], sem)
            cp3 = pltpu.async_copy(
                tab_hbm.at[idx_v.at[pl.ds(2 * b_per_w + off, ch)]],
                w_v.at[pl.ds(off, ch)], sem)
            cp4 = pltpu.async_copy(
                ab_hbm.at[idx_v.at[pl.ds(off, ch)]],
                a_v.at[pl.ds(off, ch)], sem)
            cp1.wait()
            cp2.wait()
            cp3.wait()
            cp4.wait()
            return carry

        lax.fori_loop(0, n_ch, chunk, 0)
        rows = out_hbm.at[pl.ds(base, b_per_w)]
        pltpu.sync_copy(c_v, rows.at[:, pl.ds(0, D)])
        pltpu.sync_copy(s_v, rows.at[:, pl.ds(D, D)])
        pltpu.sync_copy(w_v, rows.at[:, pl.ds(2 * D, D)])
        pltpu.sync_copy(a_v, ab_out.at[pl.ds(base, b_per_w)])

    run = pl.kernel(
        body,
        out_type=[jax.ShapeDtypeStruct((n, FD), jnp.float32),
                  jax.ShapeDtypeStruct((n,), jnp.float32)],
        mesh=mesh,
        compiler_params=pltpu.CompilerParams(use_tc_tiling_on_sc=False),
        scratch_types=[
            pltpu.VMEM((3 * b_per_w,), jnp.int32),
            pltpu.VMEM((b_per_w, D), jnp.float32),
            pltpu.VMEM((b_per_w, D), jnp.float32),
            pltpu.VMEM((b_per_w, D), jnp.float32),
            pltpu.VMEM((b_per_w,), jnp.float32),
            pltpu.SemaphoreType.DMA,
        ],
    )
    return run(ids3, tabs, ab_tab.reshape(-1))


def _call_block(Xs, Ys, W1, b1c, W2, b2, aW, abias, mask_diag, nb):
    f32 = jnp.float32
    Qs = [lax.dot_general(aW, X, (((1,), (0,)), ((), ())),
                          preferred_element_type=f32) + abias for X in Xs]
    Zs, ZQs = [], []
    for i in range(TEAM):
        for j in range(TEAM):
            Zs.append(Xs[i] * Ys[j])
            ZQs.append(Qs[i] * Ys[j])
    Z = jnp.concatenate(Zs, axis=1)
    ZQ = jnp.concatenate(ZQs, axis=1)
    H = jnp.maximum(
        lax.dot_general(W1, Z, (((1,), (0,)), ((), ())),
                        preferred_element_type=f32) + b1c, 0.0)
    E = jnp.maximum(
        lax.dot_general(W2, H, (((1,), (0,)), ((), ())),
                        preferred_element_type=f32) + b2, 0.0)
    ones_d = jnp.ones((1, D), f32)
    L = lax.dot_general(ones_d, ZQ, (((1,), (0,)), ((), ())),
                        preferred_element_type=f32)

    def seg(A, i, j):
        p = i * TEAM + j
        return A[:, p * nb:(p + 1) * nb]

    total = jnp.zeros((1, nb), f32)
    for i in range(TEAM):
        js = [j for j in range(TEAM) if not (mask_diag and j == i)]
        m = seg(L, i, js[0])
        for j in js[1:]:
            m = jnp.maximum(m, seg(L, i, j))
        exps = [jnp.exp(seg(L, i, j) - m) for j in js]
        den = exps[0]
        for ex in exps[1:]:
            den = den + ex
        rec = 1.0 / den
        for j, ex in zip(js, exps):
            total = total + seg(E, i, j) * ex * rec
    return total


def _tc_body(nb, btot, *refs):
    slot_refs = refs[:NIDS]
    (ab_ref, cW1, cb1, cW2, cb2, caW, cab,
     pW1, pb1, pW2, pb2, paW, pab, out_ref) = refs[NIDS:]
    blk = pl.program_id(0)

    bank = jnp.concatenate([r[...] for r in slot_refs], axis=0)
    bankT = jnp.transpose(bank)
    Cs = [bankT[0:D, k * nb:(k + 1) * nb] for k in range(NIDS)]
    Ss = [bankT[D:2 * D, k * nb:(k + 1) * nb] for k in range(NIDS)]
    Ws = [bankT[2 * D:3 * D, k * nb:(k + 1) * nb] for k in range(NIDS)]
    abs_ = [ab_ref[pl.ds(k * btot + blk * nb, nb)] for k in range(NIDS)]

    coop_args = (cW1[...], cb1[...], cW2[...], cb2[...], caW[...], cab[...])
    comp_args = (pW1[...], pb1[...], pW2[...], pb2[...], paW[...], pab[...])

    abA = jnp.reshape(abs_[0] + abs_[1] + abs_[2] + abs_[3] + abs_[4],
                      (1, nb))
    abB = jnp.reshape(abs_[5] + abs_[6] + abs_[7] + abs_[8] + abs_[9],
                      (1, nb))

    Sa = (abA
          + _call_block(Cs[:TEAM], Cs[:TEAM], *coop_args, True, nb)
          + _call_block(Ss[:TEAM], Ws[TEAM:], *comp_args, False, nb))
    Sb = (abB
          + _call_block(Cs[TEAM:], Cs[TEAM:], *coop_args, True, nb)
          + _call_block(Ss[TEAM:], Ws[:TEAM], *comp_args, False, nb))
    out_ref[...] = jnp.reshape(jax.nn.sigmoid(Sa - Sb), (nb,))


def _tc_compute(gat, ab_gat, coop_W1, coop_b1, coop_W2, coop_b2,
                coop_attW, coop_attb, comp_W1, comp_b1, comp_W2, comp_b2,
                comp_attW, comp_attb):
    b = gat.shape[0] // NIDS
    nb = 512
    grid = b // nb
    bpg = b // nb

    def full(shape):
        return pl.BlockSpec(shape, lambda i: (0,) * len(shape))

    slot_specs = [
        pl.BlockSpec((nb, FD), functools.partial(
            lambda k, i: (k * bpg + i, 0), k))
        for k in range(NIDS)
    ]
    in_specs = slot_specs + [
        pl.BlockSpec((b * NIDS,), lambda i: (0,)),
        full((50, D)), full((50, 1)), full((1, 50)), full((1, 1)),
        full((D, D)), full((D, 1)),
        full((50, D)), full((50, 1)), full((1, 50)), full((1, 1)),
        full((D, D)), full((D, 1)),
    ]
    out = pl.pallas_call(
        functools.partial(_tc_body, nb, b),
        grid=(grid,),
        in_specs=in_specs,
        out_specs=pl.BlockSpec((nb,), lambda i: (i,)),
        out_shape=jax.ShapeDtypeStruct((b,), jnp.float32),
    )(*([gat] * NIDS), ab_gat,
      coop_W1, coop_b1.reshape(50, 1), coop_W2, coop_b2.reshape(1, 1),
      coop_attW, coop_attb.reshape(D, 1),
      comp_W1, comp_b1.reshape(50, 1), comp_W2, comp_b2.reshape(1, 1),
      comp_attW, comp_attb.reshape(D, 1))
    return out


def kernel(two_team_hero_id, ability_tab, coop_tab, coop_W1, coop_b1,
           coop_W2, coop_b2, coop_attW, coop_attb, st_tab, wk_tab,
           comp_W1, comp_b1, comp_W2, comp_b2, comp_attW, comp_attb):
    h = coop_tab.shape[0]
    ids_sm = two_team_hero_id.astype(jnp.int32).T.reshape(-1)
    ids3 = jnp.concatenate([ids_sm, ids_sm + h, ids_sm + 2 * h])
    tabs = jnp.concatenate([coop_tab, st_tab, wk_tab], axis=0)
    gat, ab_gat = _sc_gather(ids3, tabs, ability_tab)
    return _tc_compute(gat, ab_gat, coop_W1, coop_b1, coop_W2, coop_b2,
                       coop_attW, coop_attb, comp_W1, comp_b1, comp_W2,
                       comp_b2, comp_attW, comp_attb)


# speedup vs baseline: 1.5275x; 1.5275x over previous
import functools

import jax
import jax.numpy as jnp
from jax import lax
from jax.experimental import pallas as pl
from jax.experimental.pallas import tpu as pltpu
from jax.experimental.pallas import tpu_sc as plsc

TEAM = 5
D = 32
NIDS = 2 * TEAM
FD = 96


def _sc_gather(ids_flat, coop_tab, st_tab, wk_tab, ab_tab):
    n = ids_flat.shape[0]
    info = plsc.get_sparse_core_info()
    nw = info.num_cores * info.num_subcores
    b_per_w = n // nw
    ch = 128
    n_ch = b_per_w // ch

    mesh = plsc.VectorSubcoreMesh(core_axis_name="c", subcore_axis_name="s")

    def body(idx_hbm, coop_hbm, st_hbm, wk_hbm, ab_hbm, out_hbm, ab_out,
             idx_v, c_v, s_v, w_v, a_v, sem):
        wid = lax.axis_index("s") * info.num_cores + lax.axis_index("c")
        base = wid * b_per_w
        pltpu.sync_copy(idx_hbm.at[pl.ds(base, b_per_w)], idx_v)

        def chunk(c, carry):
            off = c * ch
            idx_c = idx_v.at[pl.ds(off, ch)]
            cp1 = pltpu.async_copy(coop_hbm.at[idx_c], c_v.at[pl.ds(off, ch)], sem)
            cp2 = pltpu.async_copy(st_hbm.at[idx_c], s_v.at[pl.ds(off, ch)], sem)
            cp3 = pltpu.async_copy(wk_hbm.at[idx_c], w_v.at[pl.ds(off, ch)], sem)
            cp4 = pltpu.async_copy(ab_hbm.at[idx_c], a_v.at[pl.ds(off, ch)], sem)
            cp1.wait()
            cp2.wait()
            cp3.wait()
            cp4.wait()
            return carry

        lax.fori_loop(0, n_ch, chunk, 0)
        rows = out_hbm.at[pl.ds(base, b_per_w)]
        pltpu.sync_copy(c_v, rows.at[:, pl.ds(0, D)])
        pltpu.sync_copy(s_v, rows.at[:, pl.ds(D, D)])
        pltpu.sync_copy(w_v, rows.at[:, pl.ds(2 * D, D)])
        pltpu.sync_copy(a_v, ab_out.at[pl.ds(base, b_per_w)])

    run = pl.kernel(
        body,
        out_type=[jax.ShapeDtypeStruct((n, FD), jnp.float32),
                  jax.ShapeDtypeStruct((n,), jnp.float32)],
        mesh=mesh,
        compiler_params=pltpu.CompilerParams(use_tc_tiling_on_sc=False),
        scratch_types=[
            pltpu.VMEM((b_per_w,), jnp.int32),
            pltpu.VMEM((b_per_w, D), jnp.float32),
            pltpu.VMEM((b_per_w, D), jnp.float32),
            pltpu.VMEM((b_per_w, D), jnp.float32),
            pltpu.VMEM((b_per_w,), jnp.float32),
            pltpu.SemaphoreType.DMA,
        ],
    )
    return run(ids_flat, coop_tab, st_tab, wk_tab, ab_tab.reshape(-1))


def _call_block(Xs, Ys, W1, b1c, W2, b2, aW, abias, mask_diag, nb):
    f32 = jnp.float32
    Qs = [lax.dot_general(aW, X, (((1,), (0,)), ((), ())),
                          preferred_element_type=f32) + abias for X in Xs]
    Zs, ZQs = [], []
    for i in range(TEAM):
        for j in range(TEAM):
            Zs.append(Xs[i] * Ys[j])
            ZQs.append(Qs[i] * Ys[j])
    Z = jnp.concatenate(Zs, axis=1)
    ZQ = jnp.concatenate(ZQs, axis=1)
    H = jnp.maximum(
        lax.dot_general(W1, Z, (((1,), (0,)), ((), ())),
                        preferred_element_type=f32) + b1c, 0.0)
    E = jnp.maximum(
        lax.dot_general(W2, H, (((1,), (0,)), ((), ())),
                        preferred_element_type=f32) + b2, 0.0)
    ones_d = jnp.ones((1, D), f32)
    L = lax.dot_general(ones_d, ZQ, (((1,), (0,)), ((), ())),
                        preferred_element_type=f32)

    def seg(A, i, j):
        p = i * TEAM + j
        return A[:, p * nb:(p + 1) * nb]

    total = jnp.zeros((1, nb), f32)
    for i in range(TEAM):
        js = [j for j in range(TEAM) if not (mask_diag and j == i)]
        m = seg(L, i, js[0])
        for j in js[1:]:
            m = jnp.maximum(m, seg(L, i, j))
        exps = [jnp.exp(seg(L, i, j) - m) for j in js]
        den = exps[0]
        for ex in exps[1:]:
            den = den + ex
        rec = 1.0 / den
        for j, ex in zip(js, exps):
            total = total + seg(E, i, j) * ex * rec
    return total


def _tc_body(nb, btot, *refs):
    slot_refs = refs[:NIDS]
    (ab_ref, cW1, cb1, cW2, cb2, caW, cab,
     pW1, pb1, pW2, pb2, paW, pab, out_ref) = refs[NIDS:]
    blk = pl.program_id(0)

    bank = jnp.concatenate([r[...] for r in slot_refs], axis=0)
    bankT = jnp.transpose(bank)
    Cs = [bankT[0:D, k * nb:(k + 1) * nb] for k in range(NIDS)]
    Ss = [bankT[D:2 * D, k * nb:(k + 1) * nb] for k in range(NIDS)]
    Ws = [bankT[2 * D:3 * D, k * nb:(k + 1) * nb] for k in range(NIDS)]
    abs_ = [ab_ref[pl.ds(k * btot + blk * nb, nb)] for k in range(NIDS)]

    coop_args = (cW1[...], cb1[...], cW2[...], cb2[...], caW[...], cab[...])
    comp_args = (pW1[...], pb1[...], pW2[...], pb2[...], paW[...], pab[...])

    abA = jnp.reshape(abs_[0] + abs_[1] + abs_[2] + abs_[3] + abs_[4],
                      (1, nb))
    abB = jnp.reshape(abs_[5] + abs_[6] + abs_[7] + abs_[8] + abs_[9],
                      (1, nb))

    Sa = (abA
          + _call_block(Cs[:TEAM], Cs[:TEAM], *coop_args, True, nb)
          + _call_block(Ss[:TEAM], Ws[TEAM:], *comp_args, False, nb))
    Sb = (abB
          + _call_block(Cs[TEAM:], Cs[TEAM:], *coop_args, True, nb)
          + _call_block(Ss[TEAM:], Ws[:TEAM], *comp_args, False, nb))
    out_ref[...] = jnp.reshape(jax.nn.sigmoid(Sa - Sb), (nb,))


def _tc_compute(gat, ab_gat, coop_W1, coop_b1, coop_W2, coop_b2,
                coop_attW, coop_attb, comp_W1, comp_b1, comp_W2, comp_b2,
                comp_attW, comp_attb):
    b = gat.shape[0] // NIDS
    nb = 512
    grid = b // nb
    bpg = b // nb

    def full(shape):
        return pl.BlockSpec(shape, lambda i: (0,) * len(shape))

    slot_specs = [
        pl.BlockSpec((nb, FD), functools.partial(
            lambda k, i: (k * bpg + i, 0), k))
        for k in range(NIDS)
    ]
    in_specs = slot_specs + [
        pl.BlockSpec((b * NIDS,), lambda i: (0,)),
        full((50, D)), full((50, 1)), full((1, 50)), full((1, 1)),
        full((D, D)), full((D, 1)),
        full((50, D)), full((50, 1)), full((1, 50)), full((1, 1)),
        full((D, D)), full((D, 1)),
    ]
    out = pl.pallas_call(
        functools.partial(_tc_body, nb, b),
        grid=(grid,),
        in_specs=in_specs,
        out_specs=pl.BlockSpec((nb,), lambda i: (i,)),
        out_shape=jax.ShapeDtypeStruct((b,), jnp.float32),
    )(*([gat] * NIDS), ab_gat,
      coop_W1, coop_b1.reshape(50, 1), coop_W2, coop_b2.reshape(1, 1),
      coop_attW, coop_attb.reshape(D, 1),
      comp_W1, comp_b1.reshape(50, 1), comp_W2, comp_b2.reshape(1, 1),
      comp_attW, comp_attb.reshape(D, 1))
    return out


def kernel(two_team_hero_id, ability_tab, coop_tab, coop_W1, coop_b1,
           coop_W2, coop_b2, coop_attW, coop_attb, st_tab, wk_tab,
           comp_W1, comp_b1, comp_W2, comp_b2, comp_attW, comp_attb):
    ids_sm = two_team_hero_id.astype(jnp.int32).T.reshape(-1)
    gat, ab_gat = _sc_gather(ids_sm, coop_tab, st_tab, wk_tab, ability_tab)
    return _tc_compute(gat, ab_gat, coop_W1, coop_b1, coop_W2, coop_b2,
                       coop_attW, coop_attb, comp_W1, comp_b1, comp_W2,
                       comp_b2, comp_attW, comp_attb)
